# baseline (speedup 1.0000x reference)
import functools

import jax
import jax.numpy as jnp
from jax.experimental import pallas as pl
from jax.experimental.pallas import tpu as pltpu
from jax.experimental.pallas import tpu_sc as plsc

N = 512
E = 8192
D = 128
LP = 20


def _dot_t(a, b):
    return jax.lax.dot_general(a, b, (((1,), (1,)), ((), ())),
                               preferred_element_type=jnp.float32)


def _dot(a, b):
    return jax.lax.dot_general(a, b, (((1,), (0,)), ((), ())),
                               preferred_element_type=jnp.float32)


def _gru_body(x_ref, w_ref, b_ref, out_ref):
    PB = x_ref.shape[0]
    T = x_ref.shape[1]
    nrows = x_ref.shape[2]

    def one(x_, h, w, b):
        xh = jnp.concatenate([x_.astype(jnp.bfloat16),
                              h.astype(jnp.bfloat16)], axis=1)
        g = _dot_t(xh, w) + b
        r = jax.nn.sigmoid(g[:, 0:D])
        z = jax.nn.sigmoid(g[:, D:2 * D])
        nn_ = jnp.tanh(g[:, 2 * D:3 * D] + r * g[:, 3 * D:4 * D])
        return (1.0 - z) * nn_ + z * h

    def step(t, hs):
        xt = x_ref[:, t]
        return tuple(one(xt[i], hs[i], w_ref[i], b_ref[i])
                     for i in range(PB))

    h0 = tuple(jnp.zeros((nrows, D), jnp.float32) for _ in range(PB))
    hs = jax.lax.fori_loop(0, T, step, h0)
    for i in range(PB):
        out_ref[i] = hs[i]


def _gru_call(x_t, wbig, bbig, pb):
    P, T, n, d = x_t.shape
    return pl.pallas_call(
        _gru_body,
        grid=(P // pb,),
        in_specs=[
            pl.BlockSpec((pb, T, n, d), lambda pp: (pp, 0, 0, 0)),
            pl.BlockSpec((pb, 4 * D, 2 * D), lambda pp: (pp, 0, 0)),
            pl.BlockSpec((pb, 1, 4 * D), lambda pp: (pp, 0, 0)),
        ],
        out_specs=pl.BlockSpec((pb, n, d), lambda pp: (pp, 0, 0)),
        out_shape=jax.ShapeDtypeStruct((P, n, d), jnp.float32),
        compiler_params=pltpu.CompilerParams(
            dimension_semantics=("arbitrary",),
            vmem_limit_bytes=100 * 1024 * 1024),
    )(x_t, wbig, bbig)


def _gru_pack(w_ih, w_hh, b_ih, b_hh):
    zero = jnp.zeros_like(w_ih[..., 2 * D:3 * D, :])
    top = jnp.concatenate([w_ih[..., 0:2 * D, :], w_hh[..., 0:2 * D, :]],
                          axis=-1)
    mid = jnp.concatenate([w_ih[..., 2 * D:3 * D, :], zero], axis=-1)
    bot = jnp.concatenate([zero, w_hh[..., 2 * D:3 * D, :]], axis=-1)
    wbig = jnp.concatenate([top, mid, bot], axis=-2)
    bbig = jnp.concatenate([b_ih[..., 0:2 * D] + b_hh[..., 0:2 * D],
                            b_ih[..., 2 * D:3 * D],
                            b_hh[..., 2 * D:3 * D]], axis=-1)
    return wbig, bbig[..., None, :]


def _fc1_body(x_ref, w_ref, b_ref, o_ref):
    o_ref[...] = _dot_t(x_ref[...], w_ref[...]) + b_ref[...]


def _fc1(x, w, b):
    return pl.pallas_call(
        _fc1_body,
        out_shape=jax.ShapeDtypeStruct((N, D), jnp.float32),
    )(x, w, b.reshape(1, D))


def _fc2_body(s_ref, w_ref, b_ref, dst_ref, cw_ref, y_ref, dinv_ref):
    h0 = _dot_t(s_ref[...], w_ref[...]) + b_ref[...]

    def dchunk(c, acc):
        dv = dst_ref[pl.ds(c * 8, 8), :]
        ids = jax.lax.broadcasted_iota(jnp.int32, (N, 8, 128), 0)
        eq = (ids == dv[None, :, :]).astype(jnp.float32)
        t = jnp.sum(eq, axis=2)
        return acc + jnp.sum(t, axis=1, keepdims=True)

    deg = jax.lax.fori_loop(0, 8, dchunk, jnp.zeros((N, 1), jnp.float32))
    dinv = jax.lax.rsqrt(deg + 1.0)
    dinvb = jnp.broadcast_to(dinv, (N, D))
    dinv_ref[...] = dinvb
    y_ref[...] = _dot(h0, cw_ref[...]) * dinvb


def _fc2(s, w, b, dst_r, cw0):
    return pl.pallas_call(
        _fc2_body,
        out_shape=(
            jax.ShapeDtypeStruct((N, D), jnp.float32),
            jax.ShapeDtypeStruct((N, D), jnp.float32),
        ),
    )(s, w, b.reshape(1, D), dst_r, cw0)


def _sc_agg_body(y_hbm, src_hbm, dst_hbm, z_hbm, out_hbm,
                 sidx, didx, rows, accum_sh, sem):
    c = jax.lax.axis_index("c")
    s = jax.lax.axis_index("s")
    w = c * 16 + s

    @pl.when(s == 0)
    def _():
        pltpu.sync_copy(z_hbm, accum_sh)

    plsc.subcore_barrier()
    pltpu.sync_copy(src_hbm.at[pl.ds(w * 2, 2), :], sidx)
    pltpu.sync_copy(dst_hbm.at[pl.ds(w * 2, 2), :], didx)
    for j in range(2):
        pltpu.async_copy(y_hbm.at[sidx.at[j]], rows, sem).wait()
        pltpu.sync_copy(rows, accum_sh.at[didx.at[j]], add=True)
    plsc.subcore_barrier()

    @pl.when(s == 0)
    def _():
        pltpu.sync_copy(accum_sh, out_hbm.at[c])


def _sc_agg(y, src_r, dst_r, zrows):
    kfn = pl.kernel(
        _sc_agg_body,
        out_type=jax.ShapeDtypeStruct((2, N, D), jnp.float32),
        mesh=plsc.VectorSubcoreMesh(core_axis_name="c", subcore_axis_name="s"),
        scratch_types=[
            pltpu.VMEM((2, 128), jnp.int32),
            pltpu.VMEM((2, 128), jnp.int32),
            pltpu.VMEM((128, D), jnp.float32),
            pltpu.VMEM_SHARED((N, D), jnp.float32),
            pltpu.SemaphoreType.DMA,
        ],
    )
    return kfn(y, src_r, dst_r, zrows)


def _mid_body(p_ref, y_ref, dinv_ref, cb_ref, g_ref, lb_ref, cw_ref, o_ref):
    tot = jnp.sum(p_ref[...], axis=0)
    dinvb = dinv_ref[...]
    hn = dinvb * (tot + y_ref[...]) + cb_ref[...]
    a = jnp.maximum(hn, 0.0)
    m = jnp.mean(a, axis=1, keepdims=True)
    v = jnp.mean((a - m) * (a - m), axis=1, keepdims=True)
    ln = (a - m) * jax.lax.rsqrt(v + 1e-5) * g_ref[...] + lb_ref[...]
    o_ref[...] = _dot(ln, cw_ref[...]) * dinvb


def _mid(parts, y, dinvb, cb, g, lb, cw_next):
    return pl.pallas_call(
        _mid_body,
        out_shape=jax.ShapeDtypeStruct((N, D), jnp.float32),
    )(parts, y, dinvb, cb.reshape(1, D), g.reshape(1, D), lb.reshape(1, D),
      cw_next)


def _final_body(p_ref, y_ref, dinv_ref, cb_ref, w1_ref, b1_ref, w2_ref,
                b2_ref, emb_ref, lp_ref):
    tot = jnp.sum(p_ref[...], axis=0)
    hn = dinv_ref[...] * (tot + y_ref[...]) + cb_ref[...]
    emb_ref[...] = hn
    a = jnp.maximum(hn, 0.0)
    t = _dot_t(a, w1_ref[...]) + b1_ref[...]
    logits = _dot_t(t, w2_ref[...]) + b2_ref[...]
    mx = jnp.max(logits, axis=1, keepdims=True)
    sh = logits - mx
    lp_ref[...] = sh - jnp.log(jnp.sum(jnp.exp(sh), axis=1, keepdims=True))


def _final(parts, y, dinvb, cb, w1, b1, w2, b2):
    return pl.pallas_call(
        _final_body,
        out_shape=(
            jax.ShapeDtypeStruct((N, D), jnp.float32),
            jax.ShapeDtypeStruct((N, 16), jnp.float32),
        ),
    )(parts, y, dinvb, cb.reshape(1, D), w1, b1.reshape(1, D), w2,
      b2.reshape(1, 16))


def kernel(x, x_lp, x_ns, x_ref, x_def, x_pdt, lp_W_ih, lp_W_hh, lp_b_ih,
           lp_b_hh, lp_fc_W, lp_fc_b, ns_W_ih, ns_W_hh, ns_b_ih, ns_b_hh,
           all_fc_W, all_fc_b, conv_W, conv_b, ln_g, ln_b, mp_W1, mp_b1,
           mp_W2, mp_b2, edge_index, batch):
    n = x_ns.shape[0]

    bf = jnp.bfloat16
    xlp_t = jnp.transpose(x_lp, (1, 2, 0, 3))
    xns_t = jnp.transpose(x_ns, (1, 0, 2))[None]
    wbig_lp, bbig_lp = _gru_pack(lp_W_ih, lp_W_hh, lp_b_ih, lp_b_hh)
    wbig_lp = wbig_lp.astype(bf)
    wbig_ns, bbig_ns = _gru_pack(ns_W_ih, ns_W_hh, ns_b_ih, ns_b_hh)
    wbig_ns = wbig_ns.astype(bf)

    src_r = edge_index[0].reshape(64, 128)
    dst_r = edge_index[1].reshape(64, 128)
    zrows = jnp.zeros((n, D), jnp.float32)

    hs = _gru_call(xlp_t, wbig_lp, bbig_lp, 1)
    hns = _gru_call(xns_t, wbig_ns[None], bbig_ns[None], 1)[0]

    xlp_fc = _fc1(hs.reshape(n, D * LP), lp_fc_W, lp_fc_b)
    s640 = jnp.concatenate([x_pdt, x_ref, x_def, xlp_fc, hns],
                           axis=0).reshape(n, 5 * D)
    y0, dinvb = _fc2(s640, all_fc_W, all_fc_b, dst_r, conv_W[0])

    p0 = _sc_agg(y0, src_r, dst_r, zrows)
    y1 = _mid(p0, y0, dinvb, conv_b[0], ln_g[0], ln_b[0], conv_W[1])
    p1 = _sc_agg(y1, src_r, dst_r, zrows)
    y2 = _mid(p1, y1, dinvb, conv_b[1], ln_g[1], ln_b[1], conv_W[2])
    p2 = _sc_agg(y2, src_r, dst_r, zrows)
    emb, logp = _final(p2, y2, dinvb, conv_b[2], mp_W1, mp_b1, mp_W2, mp_b2)
    return (emb, logp)


# speedup vs baseline: 1.1086x; 1.1086x over previous
import functools

import jax
import jax.numpy as jnp
from jax.experimental import pallas as pl
from jax.experimental.pallas import tpu as pltpu
from jax.experimental.pallas import tpu_sc as plsc

N = 512
E = 8192
D = 128
LP = 20


def _dot_t(a, b):
    return jax.lax.dot_general(a, b, (((1,), (1,)), ((), ())),
                               preferred_element_type=jnp.float32)


def _dot(a, b):
    return jax.lax.dot_general(a, b, (((1,), (0,)), ((), ())),
                               preferred_element_type=jnp.float32)


def _gru_body(x_ref, wc_ref, whn_ref, bsum_ref, bhn_ref, out_ref):
    T = x_ref.shape[1]
    nrows = x_ref.shape[2]
    wc = wc_ref[0]
    whn = whn_ref[0]
    bsum = bsum_ref[0]
    bhn = bhn_ref[0]

    def step(t, h):
        xt = x_ref[0, t]
        xh = jnp.concatenate([xt, h], axis=1)
        g = _dot_t(xh, wc) + bsum
        hn = _dot_t(h, whn) + bhn
        r = jax.nn.sigmoid(g[:, 0:D])
        z = jax.nn.sigmoid(g[:, D:2 * D])
        nn_ = jnp.tanh(g[:, 2 * D:3 * D] - hn + r * hn)
        return (1.0 - z) * nn_ + z * h

    h = jax.lax.fori_loop(0, T, step, jnp.zeros((nrows, D), jnp.float32))
    out_ref[0] = h


def _gru_call(x_t, wc, whn, bsum, bhn):
    P, T, n, d = x_t.shape
    return pl.pallas_call(
        _gru_body,
        grid=(P,),
        in_specs=[
            pl.BlockSpec((1, T, n, d), lambda p: (p, 0, 0, 0)),
            pl.BlockSpec((1, 3 * D, 2 * D), lambda p: (p, 0, 0)),
            pl.BlockSpec((1, D, D), lambda p: (p, 0, 0)),
            pl.BlockSpec((1, 1, 3 * D), lambda p: (p, 0, 0)),
            pl.BlockSpec((1, 1, D), lambda p: (p, 0, 0)),
        ],
        out_specs=pl.BlockSpec((1, n, d), lambda p: (p, 0, 0)),
        out_shape=jax.ShapeDtypeStruct((P, n, d), jnp.float32),
        compiler_params=pltpu.CompilerParams(
            dimension_semantics=("arbitrary",)),
    )(x_t, wc, whn, bsum, bhn)


def _fc1_body(x_ref, w_ref, b_ref, o_ref):
    o_ref[...] = _dot_t(x_ref[...], w_ref[...]) + b_ref[...]


def _fc1(x, w, b):
    return pl.pallas_call(
        _fc1_body,
        out_shape=jax.ShapeDtypeStruct((N, D), jnp.float32),
    )(x, w, b.reshape(1, D))


def _fc2_body(s_ref, w_ref, b_ref, dst_ref, cw_ref, y_ref, dinv_ref):
    h0 = _dot_t(s_ref[...], w_ref[...]) + b_ref[...]

    def dchunk(c, acc):
        dv = dst_ref[pl.ds(c * 8, 8), :]
        ids = jax.lax.broadcasted_iota(jnp.int32, (N, 8, 128), 0)
        eq = (ids == dv[None, :, :]).astype(jnp.float32)
        t = jnp.sum(eq, axis=2)
        return acc + jnp.sum(t, axis=1, keepdims=True)

    deg = jax.lax.fori_loop(0, 8, dchunk, jnp.zeros((N, 1), jnp.float32))
    dinv = jax.lax.rsqrt(deg + 1.0)
    dinvb = jnp.broadcast_to(dinv, (N, D))
    dinv_ref[...] = dinvb
    y_ref[...] = _dot(h0, cw_ref[...]) * dinvb


def _fc2(s, w, b, dst_r, cw0):
    return pl.pallas_call(
        _fc2_body,
        out_shape=(
            jax.ShapeDtypeStruct((N, D), jnp.float32),
            jax.ShapeDtypeStruct((N, D), jnp.float32),
        ),
    )(s, w, b.reshape(1, D), dst_r, cw0)


def _sc_agg_body(y_hbm, src_hbm, dst_hbm, z_hbm, out_hbm,
                 sidx, didx, rows, accum_sh, sem):
    c = jax.lax.axis_index("c")
    s = jax.lax.axis_index("s")
    w = c * 16 + s

    @pl.when(s == 0)
    def _():
        pltpu.sync_copy(z_hbm, accum_sh)

    plsc.subcore_barrier()
    pltpu.sync_copy(src_hbm.at[pl.ds(w * 2, 2), :], sidx)
    pltpu.sync_copy(dst_hbm.at[pl.ds(w * 2, 2), :], didx)
    for j in range(2):
        pltpu.async_copy(y_hbm.at[sidx.at[j]], rows, sem).wait()
        pltpu.sync_copy(rows, accum_sh.at[didx.at[j]], add=True)
    plsc.subcore_barrier()

    @pl.when(s == 0)
    def _():
        pltpu.sync_copy(accum_sh, out_hbm.at[c])


def _sc_agg(y, src_r, dst_r, zrows):
    kfn = pl.kernel(
        _sc_agg_body,
        out_type=jax.ShapeDtypeStruct((2, N, D), jnp.float32),
        mesh=plsc.VectorSubcoreMesh(core_axis_name="c", subcore_axis_name="s"),
        scratch_types=[
            pltpu.VMEM((2, 128), jnp.int32),
            pltpu.VMEM((2, 128), jnp.int32),
            pltpu.VMEM((128, D), jnp.float32),
            pltpu.VMEM_SHARED((N, D), jnp.float32),
            pltpu.SemaphoreType.DMA,
        ],
    )
    return kfn(y, src_r, dst_r, zrows)


def _mid_body(p_ref, y_ref, dinv_ref, cb_ref, g_ref, lb_ref, cw_ref, o_ref):
    tot = jnp.sum(p_ref[...], axis=0)
    dinvb = dinv_ref[...]
    hn = dinvb * (tot + y_ref[...]) + cb_ref[...]
    a = jnp.maximum(hn, 0.0)
    m = jnp.mean(a, axis=1, keepdims=True)
    v = jnp.mean((a - m) * (a - m), axis=1, keepdims=True)
    ln = (a - m) * jax.lax.rsqrt(v + 1e-5) * g_ref[...] + lb_ref[...]
    o_ref[...] = _dot(ln, cw_ref[...]) * dinvb


def _mid(parts, y, dinvb, cb, g, lb, cw_next):
    return pl.pallas_call(
        _mid_body,
        out_shape=jax.ShapeDtypeStruct((N, D), jnp.float32),
    )(parts, y, dinvb, cb.reshape(1, D), g.reshape(1, D), lb.reshape(1, D),
      cw_next)


def _final_body(p_ref, y_ref, dinv_ref, cb_ref, w1_ref, b1_ref, w2_ref,
                b2_ref, emb_ref, lp_ref):
    tot = jnp.sum(p_ref[...], axis=0)
    hn = dinv_ref[...] * (tot + y_ref[...]) + cb_ref[...]
    emb_ref[...] = hn
    a = jnp.maximum(hn, 0.0)
    t = _dot_t(a, w1_ref[...]) + b1_ref[...]
    logits = _dot_t(t, w2_ref[...]) + b2_ref[...]
    mx = jnp.max(logits, axis=1, keepdims=True)
    sh = logits - mx
    lp_ref[...] = sh - jnp.log(jnp.sum(jnp.exp(sh), axis=1, keepdims=True))


def _final(parts, y, dinvb, cb, w1, b1, w2, b2):
    return pl.pallas_call(
        _final_body,
        out_shape=(
            jax.ShapeDtypeStruct((N, D), jnp.float32),
            jax.ShapeDtypeStruct((N, 16), jnp.float32),
        ),
    )(parts, y, dinvb, cb.reshape(1, D), w1, b1.reshape(1, D), w2,
      b2.reshape(1, 16))


def kernel(x, x_lp, x_ns, x_ref, x_def, x_pdt, lp_W_ih, lp_W_hh, lp_b_ih,
           lp_b_hh, lp_fc_W, lp_fc_b, ns_W_ih, ns_W_hh, ns_b_ih, ns_b_hh,
           all_fc_W, all_fc_b, conv_W, conv_b, ln_g, ln_b, mp_W1, mp_b1,
           mp_W2, mp_b2, edge_index, batch):
    n = x_ns.shape[0]

    xlp_t = jnp.transpose(x_lp, (1, 2, 0, 3))
    wc_lp = jnp.concatenate([lp_W_ih, lp_W_hh], axis=2)
    whn_lp = lp_W_hh[:, 2 * D:3 * D, :]
    bsum_lp = (lp_b_ih + lp_b_hh).reshape(LP, 1, 3 * D)
    bhn_lp = lp_b_hh[:, 2 * D:3 * D].reshape(LP, 1, D)

    xns_t = jnp.transpose(x_ns, (1, 0, 2))[None]
    wc_ns = jnp.concatenate([ns_W_ih, ns_W_hh], axis=1)[None]
    whn_ns = ns_W_hh[2 * D:3 * D, :][None]
    bsum_ns = (ns_b_ih + ns_b_hh).reshape(1, 1, 3 * D)
    bhn_ns = ns_b_hh[2 * D:3 * D].reshape(1, 1, D)

    src_r = edge_index[0].reshape(64, 128)
    dst_r = edge_index[1].reshape(64, 128)
    zrows = jnp.zeros((n, D), jnp.float32)

    hs = _gru_call(xlp_t, wc_lp, whn_lp, bsum_lp, bhn_lp)
    hns = _gru_call(xns_t, wc_ns, whn_ns, bsum_ns, bhn_ns)[0]

    xlp_fc = _fc1(hs.reshape(n, D * LP), lp_fc_W, lp_fc_b)
    s640 = jnp.concatenate([x_pdt, x_ref, x_def, xlp_fc, hns],
                           axis=0).reshape(n, 5 * D)
    y0, dinvb = _fc2(s640, all_fc_W, all_fc_b, dst_r, conv_W[0])

    p0 = _sc_agg(y0, src_r, dst_r, zrows)
    y1 = _mid(p0, y0, dinvb, conv_b[0], ln_g[0], ln_b[0], conv_W[1])
    p1 = _sc_agg(y1, src_r, dst_r, zrows)
    y2 = _mid(p1, y1, dinvb, conv_b[1], ln_g[1], ln_b[1], conv_W[2])
    p2 = _sc_agg(y2, src_r, dst_r, zrows)
    emb, logp = _final(p2, y2, dinvb, conv_b[2], mp_W1, mp_b1, mp_W2, mp_b2)
    return (emb, logp)
